# baseline (speedup 1.0000x reference)
import functools

import jax
import jax.numpy as jnp
from jax import lax
from jax.experimental import pallas as pl
from jax.experimental.pallas import tpu as pltpu
from jax.experimental.pallas import tpu_sc as plsc

_N = 100000
_NN = _N + 1
_B = 32
_E = 1600000
_NC = 2
_NS = 16
_L = 16

_EPAD = 1646592
_ROWS = _EPAD // 128
_RPW = _ROWS // _NS
_CR = 2
_CHUNKS = _RPW // _CR
_NB = 3

_WPN = 6256
_NP = _WPN * _NS

_mesh = plsc.VectorSubcoreMesh(core_axis_name="c", subcore_axis_name="s")

_set = lambda: [
    pltpu.VMEM((_CR, 3, 128), jnp.int32),
    pltpu.VMEM((_CR, 128, _L), jnp.float32),
    pltpu.VMEM((_CR, 128, _L), jnp.float32),
]

_SEMS = [pltpu.SemaphoreType.DMA] * (3 * _NB)


@jax.jit
def _circuit_sc(xtr, epk):
    @functools.partial(
        pl.kernel,
        out_type=jax.ShapeDtypeStruct((_NC, _NP, _L), jnp.float32),
        mesh=_mesh,
        scratch_types=sum([_set() for _ in range(_NB)], []) + [
            pltpu.VMEM_SHARED((_NP, _L), jnp.float32),
        ] + _SEMS,
        compiler_params=pltpu.CompilerParams(use_tc_tiling_on_sc=False,
                                             needs_layout_passes=False),
    )
    def k(xtr_hbm, epk_hbm, out_hbm, *rest):
        bufs = []
        for p in range(_NB):
            bufs.append(tuple(rest[3 * p: 3 * p + 3]))
        acc = rest[3 * _NB]
        sems = rest[3 * _NB + 1:]
        sets = tuple(bufs[p] + (sems[3 * p], sems[3 * p + 1], sems[3 * p + 2])
                     for p in range(_NB))

        c = lax.axis_index("c")
        s = lax.axis_index("s")
        vs0 = sets[0][1]

        for j in range(_CR):
            @pl.loop(0, 128, unroll=8)
            def _(i, j=j):
                vs0[j, i] = jnp.zeros((_L,), jnp.float32)

        zbase = s * _WPN
        _ZC = _WPN // (_CR * 128)
        _zrem = _WPN % (_CR * 128)
        for tk in range(_ZC):
            for j in range(_CR):
                pltpu.sync_copy(
                    vs0.at[j],
                    acc.at[pl.ds(zbase + (tk * _CR + j) * 128, 128)])
        if _zrem:
            pltpu.sync_copy(vs0.at[0, pl.ds(0, _zrem)],
                            acc.at[pl.ds(zbase + _ZC * _CR * 128, _zrem)])
        plsc.subcore_barrier()

        row_base = s * _RPW
        table = xtr_hbm.at[c]

        def fire_load(chunk, p):
            (eb, _, _, lsem, _, _) = sets[p]
            off = row_base + chunk * _CR
            pltpu.async_copy(epk_hbm.at[pl.ds(off, _CR)], eb, lsem)

        def drain_load(p):
            (eb, _, _, lsem, _, _) = sets[p]
            pltpu.make_async_copy(epk_hbm.at[pl.ds(0, _CR)], eb, lsem).wait()

        def fire_gathers(p):
            (eb, vsb, vdb, _, gsem, _) = sets[p]
            for j in range(_CR):
                pltpu.async_copy(table.at[eb.at[j, 0]], vsb.at[j], gsem)
                pltpu.async_copy(table.at[eb.at[j, 1]], vdb.at[j], gsem)

        def drain_gathers(p):
            (eb, vsb, vdb, _, gsem, _) = sets[p]
            for j in range(_CR):
                pltpu.make_async_copy(table.at[eb.at[j, 0]],
                                      vsb.at[j], gsem).wait()
                pltpu.make_async_copy(table.at[eb.at[j, 1]],
                                      vdb.at[j], gsem).wait()

        def compute_and_fire_scatters(p):
            (eb, vsb, vdb, _, _, ssem) = sets[p]
            for r in range(_CR):
                @plsc.parallel_loop(0, 128, 16)
                def _(e0, r=r):
                    pv = plsc.bitcast(eb[r, 2, pl.ds(e0, _L)], jnp.float32)
                    for i in range(_L):
                        e = e0 + i
                        row = pv[i] * (vsb[r, e] - vdb[r, e])
                        vsb[r, e] = row
                        vdb[r, e] = -row
            for j in range(_CR):
                pltpu.async_copy(vdb.at[j], acc.at[eb.at[j, 0]], ssem,
                                 add=True)
                pltpu.async_copy(vsb.at[j], acc.at[eb.at[j, 1]], ssem,
                                 add=True)

        def drain_scatters(p):
            (eb, vsb, vdb, _, _, ssem) = sets[p]
            for j in range(_CR):
                pltpu.make_async_copy(vdb.at[j], acc.at[eb.at[j, 0]],
                                      ssem).wait()
                pltpu.make_async_copy(vsb.at[j], acc.at[eb.at[j, 1]],
                                      ssem).wait()

        def phase(p, k_prev_next, *, drain_sc=True, fire_ld=True,
                  fire_ga=True):
            prev = (p + _NB - 1) % _NB
            q2 = (p + _NB - 2) % _NB
            if drain_sc:
                drain_scatters(prev)
            if fire_ld:
                fire_load(k_prev_next, prev)
            if fire_ga:
                drain_load(q2)
                fire_gathers(q2)
            drain_gathers(p)
            compute_and_fire_scatters(p)

        for p in range(_NB - 1):
            fire_load(p, p)
        for p in range(_NB - 2):
            drain_load(p)
            fire_gathers(p)

        phase(0, _NB - 1, drain_sc=False)
        for p in range(1, _NB):
            phase(p, p - 1 + _NB)

        @pl.loop(_NB, _CHUNKS - _NB, step=_NB)
        def _(base):
            for p in range(_NB):
                phase(p, base + p - 1 + _NB)

        phase(0, _CHUNKS - 1)
        phase(1, 0, fire_ld=False)
        for p in range(2, _NB):
            phase(p, 0, fire_ld=False, fire_ga=False)
        drain_scatters(_NB - 1)

        plsc.subcore_barrier()

        wbase = s * _WPN
        for tk in range(_ZC):
            for j in range(_CR):
                pltpu.sync_copy(
                    acc.at[pl.ds(wbase + (tk * _CR + j) * 128, 128)],
                    out_hbm.at[c, pl.ds(wbase + (tk * _CR + j) * 128, 128)])
        if _zrem:
            pltpu.sync_copy(
                acc.at[pl.ds(wbase + _ZC * _CR * 128, _zrem)],
                out_hbm.at[c, pl.ds(wbase + _ZC * _CR * 128, _zrem)])

    return k(xtr, epk)


def kernel(t, x, src, des, param):
    del t
    aux_t = jnp.concatenate([jnp.zeros((1, _B), x.dtype), x.T], axis=0)
    xtr = aux_t.reshape(_NN, _NC, _L).transpose(1, 0, 2)

    pad = _EPAD - _E
    pad_idx = (jnp.arange(pad, dtype=jnp.int32) % _N) + 1
    src2 = jnp.concatenate([src, pad_idx]).reshape(_ROWS, 128)
    des2 = jnp.concatenate([des, pad_idx]).reshape(_ROWS, 128)
    par2 = jax.lax.bitcast_convert_type(
        jnp.concatenate([param, jnp.zeros((pad,), param.dtype)]),
        jnp.int32).reshape(_ROWS, 128)
    epk = jnp.stack([src2, des2, par2], axis=1)

    out = _circuit_sc(xtr, epk)
    res = jnp.concatenate([out[0, 1:_NN, :], out[1, 1:_NN, :]], axis=-1)
    return res.T


# speedup vs baseline: 1.2099x; 1.2099x over previous
import functools

import jax
import jax.numpy as jnp
from jax import lax
from jax.experimental import pallas as pl
from jax.experimental.pallas import tpu as pltpu
from jax.experimental.pallas import tpu_sc as plsc

_N = 100000
_NN = _N + 1
_B = 32
_E = 1600000
_NC = 2
_NS = 16
_L = 16

_EPAD = 1638400
_ROWS = _EPAD // 128
_RPW = _ROWS // _NS
_CR = 1
_CHUNKS = _RPW // _CR
_CE = _CR * 128
_NB = 4

_WPN = 6256
_NP = _WPN * _NS

_mesh = plsc.VectorSubcoreMesh(core_axis_name="c", subcore_axis_name="s")

_set = lambda: [
    pltpu.VMEM((_CR, 128), jnp.int32),
    pltpu.VMEM((_CR, 128), jnp.int32),
    pltpu.VMEM((_CR, 128), jnp.float32),
    pltpu.VMEM((_CE, _L), jnp.float32),
    pltpu.VMEM((_CE, _L), jnp.float32),
]

_SEMS = [pltpu.SemaphoreType.DMA] * (3 * _NB)


@jax.jit
def _circuit_sc(xtr, src2, des2, par2):
    @functools.partial(
        pl.kernel,
        out_type=jax.ShapeDtypeStruct((_NC, _NP, _L), jnp.float32),
        mesh=_mesh,
        scratch_types=sum([_set() for _ in range(_NB)], []) + [
            pltpu.VMEM_SHARED((_NP, _L), jnp.float32),
        ] + _SEMS,
        compiler_params=pltpu.CompilerParams(use_tc_tiling_on_sc=False),
    )
    def k(xtr_hbm, src_hbm, des_hbm, par_hbm, out_hbm, *rest):
        bufs = []
        for p in range(_NB):
            bufs.append(tuple(rest[5 * p: 5 * p + 5]))
        acc = rest[5 * _NB]
        sems = rest[5 * _NB + 1:]
        sets = tuple(bufs[p] + (sems[3 * p], sems[3 * p + 1], sems[3 * p + 2])
                     for p in range(_NB))

        c = lax.axis_index("c")
        s = lax.axis_index("s")
        vs0 = sets[0][3]

        @pl.loop(0, _CE, unroll=8)
        def _(i):
            vs0[i] = jnp.zeros((_L,), jnp.float32)

        zbase = s * _WPN
        for tk in range(_WPN // _CE):
            pltpu.sync_copy(vs0.at[pl.ds(0, _CE)],
                            acc.at[pl.ds(zbase + tk * _CE, _CE)])
        _rem = _WPN % _CE
        if _rem:
            pltpu.sync_copy(vs0.at[pl.ds(0, _rem)],
                            acc.at[pl.ds(zbase + (_WPN // _CE) * _CE, _rem)])
        plsc.subcore_barrier()

        row_base = s * _RPW
        table = xtr_hbm.at[c]

        def fire_loads(chunk, p):
            (srcb, desb, parb, _, _, lsem, _, _) = sets[p]
            off = row_base + chunk * _CR
            pltpu.async_copy(src_hbm.at[pl.ds(off, _CR)], srcb, lsem)
            pltpu.async_copy(des_hbm.at[pl.ds(off, _CR)], desb, lsem)
            pltpu.async_copy(par_hbm.at[pl.ds(off, _CR)], parb, lsem)

        def drain_loads(p):
            (srcb, desb, parb, _, _, lsem, _, _) = sets[p]
            pltpu.make_async_copy(src_hbm.at[pl.ds(0, _CR)], srcb, lsem).wait()
            pltpu.make_async_copy(des_hbm.at[pl.ds(0, _CR)], desb, lsem).wait()
            pltpu.make_async_copy(par_hbm.at[pl.ds(0, _CR)], parb, lsem).wait()

        def fire_gathers(p):
            (srcb, desb, _, vsb, vdb, _, gsem, _) = sets[p]
            for j in range(_CR):
                pltpu.async_copy(table.at[srcb.at[j]],
                                 vsb.at[pl.ds(j * 128, 128)], gsem)
                pltpu.async_copy(table.at[desb.at[j]],
                                 vdb.at[pl.ds(j * 128, 128)], gsem)

        def drain_gathers(p):
            (srcb, desb, _, vsb, vdb, _, gsem, _) = sets[p]
            for j in range(_CR):
                pltpu.make_async_copy(table.at[srcb.at[j]],
                                      vsb.at[pl.ds(j * 128, 128)], gsem).wait()
                pltpu.make_async_copy(table.at[desb.at[j]],
                                      vdb.at[pl.ds(j * 128, 128)], gsem).wait()

        def compute_and_fire_scatters(p):
            (srcb, desb, parb, vsb, vdb, _, _, ssem) = sets[p]
            for r in range(_CR):
                @plsc.parallel_loop(0, 128, 16)
                def _(e0, r=r):
                    pv = parb[r, pl.ds(e0, _L)]
                    for i in range(_L):
                        e = r * 128 + e0 + i
                        row = pv[i] * (vsb[e] - vdb[e])
                        vsb[e] = row
                        vdb[e] = -row
            for j in range(_CR):
                pltpu.async_copy(vdb.at[pl.ds(j * 128, 128)],
                                 acc.at[srcb.at[j]], ssem, add=True)
                pltpu.async_copy(vsb.at[pl.ds(j * 128, 128)],
                                 acc.at[desb.at[j]], ssem, add=True)

        def drain_scatters(p):
            (srcb, desb, _, vsb, vdb, _, _, ssem) = sets[p]
            for j in range(_CR):
                pltpu.make_async_copy(vdb.at[pl.ds(j * 128, 128)],
                                      acc.at[srcb.at[j]], ssem).wait()
                pltpu.make_async_copy(vsb.at[pl.ds(j * 128, 128)],
                                      acc.at[desb.at[j]], ssem).wait()

        def phase(p, k_prev_next, k_gather, *, drain_sc=True,
                  fire_ld=True, fire_ga=True):
            prev = (p + _NB - 1) % _NB
            q2 = (p + _NB - 2) % _NB
            if drain_sc:
                drain_scatters(prev)
            if fire_ld:
                fire_loads(k_prev_next, prev)
            if fire_ga:
                drain_loads(q2)
                fire_gathers(q2)
            drain_gathers(p)
            compute_and_fire_scatters(p)

        for p in range(_NB - 1):
            fire_loads(p, p)
        for p in range(2):
            drain_loads(p)
            fire_gathers(p)

        phase(0, _NB - 1, 0, drain_sc=False)
        for p in range(1, _NB):
            phase(p, p - 1 + _NB, p)

        @pl.loop(_NB, _CHUNKS - _NB, step=_NB)
        def _(base):
            for p in range(_NB):
                phase(p, base + p - 1 + _NB, base + p)

        phase(0, _CHUNKS - 1, _CHUNKS - _NB)
        phase(1, 0, _CHUNKS - _NB + 1, fire_ld=False)
        for p in range(2, _NB):
            phase(p, 0, 0, fire_ld=False, fire_ga=False)
        drain_scatters(_NB - 1)

        plsc.subcore_barrier()

        wbase = s * _WPN
        for tk in range(_WPN // _CE):
            pltpu.sync_copy(acc.at[pl.ds(wbase + tk * _CE, _CE)],
                            out_hbm.at[c, pl.ds(wbase + tk * _CE, _CE)])
        if _rem:
            pltpu.sync_copy(acc.at[pl.ds(wbase + (_WPN // _CE) * _CE, _rem)],
                            out_hbm.at[c, pl.ds(wbase + (_WPN // _CE) * _CE, _rem)])

    return k(xtr, src2, des2, par2)


def kernel(t, x, src, des, param):
    del t
    aux_t = jnp.concatenate([jnp.zeros((1, _B), x.dtype), x.T], axis=0)
    xtr = aux_t.reshape(_NN, _NC, _L).transpose(1, 0, 2)

    pad = _EPAD - _E
    pad_idx = (jnp.arange(pad, dtype=jnp.int32) % _N) + 1
    src2 = jnp.concatenate([src, pad_idx]).reshape(_ROWS, 128)
    des2 = jnp.concatenate([des, pad_idx]).reshape(_ROWS, 128)
    par2 = jnp.concatenate([param, jnp.zeros((pad,), param.dtype)]).reshape(_ROWS, 128)

    out = _circuit_sc(xtr, src2, des2, par2)
    res = jnp.concatenate([out[0, 1:_NN, :], out[1, 1:_NN, :]], axis=-1)
    return res.T
